# Initial kernel scaffold; baseline (speedup 1.0000x reference)
import jax
import jax.numpy as jnp
from jax.experimental import pallas as pl


def kernel(edge_index, edge_vals, uEmbeds, iEmbeds, keepRate):
    raise NotImplementedError("write your pallas kernel here")


# speedup vs baseline: 4.0564x; 4.0564x over previous
import functools

import jax
import jax.numpy as jnp
from jax import lax
from jax.experimental import pallas as pl
from jax.experimental.pallas import tpu as pltpu
from jax.experimental.pallas import tpu_sc as plsc

_USER = 5000
_ITEM = 5000
_N = _USER + _ITEM
_E = 320000
_D = 128
_LAYERS = 3

_NC = 2
_NS = 16
_NW = _NC * _NS
_B = 128
_RPT = (_N // _NS) // 8 * 8
_REM = _N - _NS * _RPT


def _make_sc_layer(C):
    mesh = plsc.VectorSubcoreMesh(core_axis_name="c", subcore_axis_name="s",
                                  num_cores=_NC, num_subcores=_NS)

    @functools.partial(
        pl.kernel,
        out_type=jax.ShapeDtypeStruct((_NC, _N, _D), jnp.float32),
        mesh=mesh,
        scratch_types=[
            pltpu.VMEM((C, _B), jnp.int32),
            pltpu.VMEM((C, _B), jnp.int32),
            pltpu.VMEM((C, _B), jnp.float32),
            pltpu.VMEM((_B, _D), jnp.float32),
            pltpu.VMEM_SHARED((_N, _D), jnp.float32),
            pltpu.SemaphoreType.DMA,
        ],
    )
    def sc_layer(x_hbm, cols_hbm, rows_hbm, vals_hbm, zeros_hbm, out_hbm,
                 cols_v, rows_v, vals_v, gbuf, acc, sem):
        c = lax.axis_index("c")
        s = lax.axis_index("s")
        wid = c * _NS + s
        r0 = s * _RPT
        pltpu.sync_copy(zeros_hbm.at[pl.ds(r0, _RPT)], acc.at[pl.ds(r0, _RPT)])

        @pl.when(s == 0)
        def _():
            pltpu.sync_copy(zeros_hbm.at[pl.ds(_NS * _RPT, _REM)],
                            acc.at[pl.ds(_NS * _RPT, _REM)])
        pltpu.sync_copy(cols_hbm.at[wid], cols_v)
        pltpu.sync_copy(rows_hbm.at[wid], rows_v)
        pltpu.sync_copy(vals_hbm.at[wid], vals_v)
        plsc.subcore_barrier()

        def chunk(j, carry):
            pltpu.async_copy(x_hbm.at[cols_v.at[j]], gbuf, sem).wait()

            def group(g, carry2):
                v16 = vals_v[j, pl.ds(g * 16, 16)]
                for k in range(16):
                    e = g * 16 + k
                    v = _splat_lane(v16, k)
                    for d in range(_D // 16):
                        sl = pl.ds(d * 16, 16)
                        gbuf[e, sl] = gbuf[e, sl] * v
                return carry2

            lax.fori_loop(0, _B // 16, group, 0)
            pltpu.sync_copy(gbuf, acc.at[rows_v.at[j]], add=True)
            return carry

        lax.fori_loop(0, C, chunk, 0)

        plsc.subcore_barrier()
        pltpu.sync_copy(acc.at[pl.ds(r0, _RPT)],
                        out_hbm.at[c, pl.ds(r0, _RPT)])

        @pl.when(s == 0)
        def _():
            pltpu.sync_copy(acc.at[pl.ds(_NS * _RPT, _REM)],
                            out_hbm.at[c, pl.ds(_NS * _RPT, _REM)])

    return sc_layer


def _splat_lane(v16, k):
    return lax.gather(
        v16,
        jnp.full((16, 1), k, jnp.int32),
        lax.GatherDimensionNumbers(
            offset_dims=(), collapsed_slice_dims=(0,), start_index_map=(0,)),
        slice_sizes=(1,),
        mode=lax.GatherScatterMode.PROMISE_IN_BOUNDS,
    )


_RB = 400


def _tc_combine_body(p0_ref, p1_ref, t_ref, x_out, t_out):
    x = p0_ref[...] + p1_ref[...]
    x_out[...] = x
    t_out[...] = t_ref[...] + x


def _tc_combine(p0, p1, t_in):
    bs = lambda: pl.BlockSpec((_RB, _D), lambda i: (i, 0))
    return pl.pallas_call(
        _tc_combine_body,
        grid=(_N // _RB,),
        in_specs=[bs(), bs(), bs()],
        out_specs=[bs(), bs()],
        out_shape=[jax.ShapeDtypeStruct((_N, _D), jnp.float32)] * 2,
    )(p0, p1, t_in)


def kernel(edge_index, edge_vals, uEmbeds, iEmbeds, keepRate):
    rows = edge_index[0]
    cols = edge_index[1]
    C = pl.cdiv(_E, _NW * _B)
    pad = _NW * _B * C - _E
    rows3 = jnp.reshape(jnp.pad(rows, (0, pad)), (_NW, C, _B))
    cols3 = jnp.reshape(jnp.pad(cols, (0, pad)), (_NW, C, _B))
    vals3 = jnp.reshape(jnp.pad(edge_vals, (0, pad)), (_NW, C, _B))
    x = jnp.concatenate([uEmbeds, iEmbeds], axis=0)
    zeros = jnp.zeros((_N, _D), jnp.float32)
    sc_layer = _make_sc_layer(C)
    total = x
    for _ in range(_LAYERS):
        partials = sc_layer(x, cols3, rows3, vals3, zeros)
        x, total = _tc_combine(partials[0], partials[1], total)
    return total[:_USER], total[_USER:]
